# Initial kernel scaffold; baseline (speedup 1.0000x reference)
import jax
import jax.numpy as jnp
from jax.experimental import pallas as pl


def kernel(x, batch_idx, attn_tensor, w_agg, b_agg, W1, att_src1, att_dst1, We1, att_e1, b1, W2, att_src2, att_dst2, We2, att_e2, b2, fc_w, fc_b):
    raise NotImplementedError("write your pallas kernel here")


# speedup vs baseline: 3459.0260x; 3459.0260x over previous
import jax
import jax.numpy as jnp
from jax import lax
from jax.experimental import pallas as pl
from jax.experimental.pallas import tpu as pltpu

N = 1024
F_IN = 128
HID = 128
HEADS = 4
OUT = 16
NG = 16
NCH = 12

TJ = 128
NJ = N // TJ

_NEG = -1e30


def _attn_tile(agg, mask, a_src, a_dst, we_c, xs_h):
    alpha = a_src + a_dst + agg * we_c
    alpha = jnp.where(alpha >= 0.0, alpha, 0.2 * alpha)
    alpha = jnp.where(mask, alpha, _NEG)
    m = jnp.max(alpha, axis=0, keepdims=True)
    p = jnp.where(mask, jnp.exp(alpha - m), 0.0)
    s = jnp.sum(p, axis=0, keepdims=True)
    p = p / (s + 1e-16)
    return lax.dot_general(p, xs_h, (((0,), (0,)), ((), ())),
                           preferred_element_type=jnp.float32)


def _layer1_kernel(attn_ref, x_ref, w_agg_ref, b_agg_ref, W1_ref,
                   asrc_ref, adst_ref, We_ref, ae_ref, b1_ref,
                   agg_out_ref, h_out_ref, xs_scr):
    j = pl.program_id(0)

    @pl.when(j == 0)
    def _():
        xs_scr[...] = jnp.dot(x_ref[...], W1_ref[...],
                              preferred_element_type=jnp.float32)

    agg = attn_ref[0] * w_agg_ref[0:1, 0:1]
    for k in range(1, NCH):
        agg = agg + attn_ref[k] * w_agg_ref[0:1, k:k + 1]
    agg = agg + b_agg_ref[0:1, 0:1]
    agg_out_ref[...] = agg
    mask = agg > 0.0

    xs = xs_scr[...]
    xs_dst = xs_scr[pl.ds(j * TJ, TJ), :]
    for h in range(HEADS):
        c0 = h * HID
        xs_h = xs[:, c0:c0 + HID]
        a_src = lax.dot_general(xs_h, asrc_ref[h:h + 1, :],
                                (((1,), (1,)), ((), ())),
                                preferred_element_type=jnp.float32)
        a_dst = lax.dot_general(adst_ref[h:h + 1, :], xs_dst[:, c0:c0 + HID],
                                (((1,), (1,)), ((), ())),
                                preferred_element_type=jnp.float32)
        we_c = jnp.sum(We_ref[0:1, c0:c0 + HID] * ae_ref[h:h + 1, :],
                       axis=1, keepdims=True)
        out_h = _attn_tile(agg, mask, a_src, a_dst, we_c, xs_h)
        h_out_ref[:, c0:c0 + HID] = jnp.maximum(
            out_h + b1_ref[0:1, c0:c0 + HID], 0.0)


def _layer2_kernel(agg_ref, h_ref, W2_ref, asrc_ref, adst_ref, We_ref,
                   ae_ref, b2_ref, batch_ref, fcw_ref, fcb_ref,
                   out_ref, xs_scr, h2_scr):
    j = pl.program_id(0)

    @pl.when(j == 0)
    def _():
        xs_scr[...] = jnp.dot(h_ref[...], W2_ref[...],
                              preferred_element_type=jnp.float32)

    agg = agg_ref[...]
    mask = agg > 0.0
    xs = xs_scr[...]
    a_src = lax.dot_general(xs, asrc_ref[...], (((1,), (1,)), ((), ())),
                            preferred_element_type=jnp.float32)
    xs_dst = xs_scr[pl.ds(j * TJ, TJ), :]
    a_dst = lax.dot_general(adst_ref[...], xs_dst, (((1,), (1,)), ((), ())),
                            preferred_element_type=jnp.float32)
    we_c = jnp.sum(We_ref[...] * ae_ref[...], axis=1, keepdims=True)
    out_t = _attn_tile(agg, mask, a_src, a_dst, we_c, xs)
    h2_scr[pl.ds(j * TJ, TJ), :] = out_t + b2_ref[...]

    @pl.when(j == NJ - 1)
    def _():
        groups = lax.broadcasted_iota(jnp.int32, (NG, N), 0)
        onehot = jnp.where(groups == batch_ref[...], 1.0, 0.0)
        sums = jnp.dot(onehot, h2_scr[...],
                       preferred_element_type=jnp.float32)
        counts = jnp.sum(onehot, axis=1, keepdims=True)
        pooled = sums / jnp.maximum(counts, 1.0)
        logits = jnp.dot(pooled, fcw_ref[...],
                         preferred_element_type=jnp.float32) + fcb_ref[...]
        mx = jnp.max(logits, axis=1, keepdims=True)
        z = logits - mx
        lse = jnp.log(jnp.sum(jnp.exp(z), axis=1, keepdims=True))
        out_ref[...] = z - lse


def kernel(x, batch_idx, attn_tensor, w_agg, b_agg,
           W1, att_src1, att_dst1, We1, att_e1, b1,
           W2, att_src2, att_dst2, We2, att_e2, b2,
           fc_w, fc_b):
    w_agg2 = jnp.reshape(w_agg.astype(jnp.float32), (1, NCH))
    b_agg2 = jnp.reshape(b_agg.astype(jnp.float32), (1, 1))
    b1_2 = jnp.reshape(b1, (1, HEADS * HID))
    b2_2 = jnp.reshape(b2, (1, HID))
    fcb2 = jnp.reshape(fc_b, (1, OUT))
    batch2 = jnp.reshape(batch_idx.astype(jnp.int32), (1, N))

    full = lambda shape: pl.BlockSpec(shape, lambda j: (0,) * len(shape))

    agg_mat, h1 = pl.pallas_call(
        _layer1_kernel,
        grid=(NJ,),
        in_specs=[
            pl.BlockSpec((NCH, N, TJ), lambda j: (0, 0, j)),
            full((N, F_IN)),
            full((1, NCH)),
            full((1, 1)),
            full((F_IN, HEADS * HID)),
            full((HEADS, HID)),
            full((HEADS, HID)),
            full((1, HEADS * HID)),
            full((HEADS, HID)),
            full((1, HEADS * HID)),
        ],
        out_specs=[
            pl.BlockSpec((N, TJ), lambda j: (0, j)),
            pl.BlockSpec((TJ, HEADS * HID), lambda j: (j, 0)),
        ],
        out_shape=[
            jax.ShapeDtypeStruct((N, N), jnp.float32),
            jax.ShapeDtypeStruct((N, HEADS * HID), jnp.float32),
        ],
        scratch_shapes=[pltpu.VMEM((N, HEADS * HID), jnp.float32)],
        compiler_params=pltpu.CompilerParams(
            dimension_semantics=("arbitrary",)),
    )(attn_tensor, x, w_agg2, b_agg2, W1, att_src1, att_dst1, We1,
      att_e1, b1_2)

    out = pl.pallas_call(
        _layer2_kernel,
        grid=(NJ,),
        in_specs=[
            pl.BlockSpec((N, TJ), lambda j: (0, j)),
            full((N, HEADS * HID)),
            full((HEADS * HID, HID)),
            full((1, HID)),
            full((1, HID)),
            full((1, HID)),
            full((1, HID)),
            full((1, HID)),
            full((1, N)),
            full((HID, OUT)),
            full((1, OUT)),
        ],
        out_specs=pl.BlockSpec((NG, OUT), lambda j: (0, 0)),
        out_shape=jax.ShapeDtypeStruct((NG, OUT), jnp.float32),
        scratch_shapes=[
            pltpu.VMEM((N, HID), jnp.float32),
            pltpu.VMEM((N, HID), jnp.float32),
        ],
        compiler_params=pltpu.CompilerParams(
            dimension_semantics=("arbitrary",)),
    )(agg_mat, h1, W2, att_src2, att_dst2, We2, att_e2, b2_2, batch2,
      fc_w, fcb2)
    return out


# speedup vs baseline: 4533.5600x; 1.3106x over previous
import jax
import jax.numpy as jnp
from jax import lax
from jax.experimental import pallas as pl
from jax.experimental.pallas import tpu as pltpu

N = 1024
F_IN = 128
HID = 128
HEADS = 4
OUT = 16
NG = 16
NCH = 12

TJ1 = 256
NJ1 = N // TJ1
TJ2 = 512
NJ2 = N // TJ2

_NEG = -1e30


def _attn_tile(agg, mask, a_src, a_dst, we_c, xs_h):
    alpha = a_src + a_dst + agg * we_c
    alpha = jnp.maximum(alpha, 0.2 * alpha)
    alpha = jnp.where(mask, alpha, _NEG)
    m = jnp.max(alpha, axis=0, keepdims=True)
    p = jnp.exp(alpha - m)
    s = jnp.sum(p, axis=0, keepdims=True)
    denom = jnp.where(m > 0.5 * _NEG, s + 1e-16, jnp.inf)
    p = p * (1.0 / denom)
    return lax.dot_general(p, xs_h, (((0,), (0,)), ((), ())),
                           preferred_element_type=jnp.float32)


def _layer1_kernel(attn_ref, x_ref, w_agg_ref, b_agg_ref, W1_ref,
                   asrc_ref, adst_ref, We_ref, ae_ref, b1_ref,
                   agg_out_ref, h_out_ref, xs_scr, as_scr, ad_scr):
    j = pl.program_id(0)

    @pl.when(j == 0)
    def _():
        xs = jnp.dot(x_ref[...], W1_ref[...],
                     preferred_element_type=jnp.float32)
        xs_scr[...] = xs
        for h in range(HEADS):
            c0 = h * HID
            xs_h = xs[:, c0:c0 + HID]
            as_scr[:, h:h + 1] = lax.dot_general(
                xs_h, asrc_ref[h:h + 1, :], (((1,), (1,)), ((), ())),
                preferred_element_type=jnp.float32)
            ad_scr[h:h + 1, :] = lax.dot_general(
                adst_ref[h:h + 1, :], xs_h, (((1,), (1,)), ((), ())),
                preferred_element_type=jnp.float32)

    acc0 = attn_ref[0] * w_agg_ref[0:1, 0:1]
    acc1 = attn_ref[1] * w_agg_ref[0:1, 1:2]
    for k in range(2, NCH, 2):
        acc0 = acc0 + attn_ref[k] * w_agg_ref[0:1, k:k + 1]
        acc1 = acc1 + attn_ref[k + 1] * w_agg_ref[0:1, k + 1:k + 2]
    agg = acc0 + acc1 + b_agg_ref[0:1, 0:1]
    agg_out_ref[...] = agg
    mask = agg > 0.0

    xs = xs_scr[...]
    for h in range(HEADS):
        c0 = h * HID
        xs_h = xs[:, c0:c0 + HID]
        a_src = as_scr[:, h:h + 1]
        a_dst = ad_scr[h:h + 1, pl.ds(j * TJ1, TJ1)]
        we_c = jnp.sum(We_ref[0:1, c0:c0 + HID] * ae_ref[h:h + 1, :],
                       axis=1, keepdims=True)
        out_h = _attn_tile(agg, mask, a_src, a_dst, we_c, xs_h)
        h_out_ref[:, c0:c0 + HID] = jnp.maximum(
            out_h + b1_ref[0:1, c0:c0 + HID], 0.0)


def _layer2_kernel(agg_ref, h_ref, W2_ref, asrc_ref, adst_ref, We_ref,
                   ae_ref, b2_ref, batch_ref, fcw_ref, fcb_ref,
                   out_ref, xs_scr, h2_scr, as_scr, ad_scr):
    j = pl.program_id(0)

    @pl.when(j == 0)
    def _():
        xs = jnp.dot(h_ref[...], W2_ref[...],
                     preferred_element_type=jnp.float32)
        xs_scr[...] = xs
        as_scr[:, 0:1] = lax.dot_general(
            xs, asrc_ref[...], (((1,), (1,)), ((), ())),
            preferred_element_type=jnp.float32)
        ad_scr[0:1, :] = lax.dot_general(
            adst_ref[...], xs, (((1,), (1,)), ((), ())),
            preferred_element_type=jnp.float32)

    agg = agg_ref[...]
    mask = agg > 0.0
    xs = xs_scr[...]
    a_src = as_scr[:, 0:1]
    a_dst = ad_scr[0:1, pl.ds(j * TJ2, TJ2)]
    we_c = jnp.sum(We_ref[...] * ae_ref[...], axis=1, keepdims=True)
    out_t = _attn_tile(agg, mask, a_src, a_dst, we_c, xs)
    h2_scr[pl.ds(j * TJ2, TJ2), :] = out_t + b2_ref[...]

    @pl.when(j == NJ2 - 1)
    def _():
        groups = lax.broadcasted_iota(jnp.int32, (NG, N), 0)
        onehot = jnp.where(groups == batch_ref[...], 1.0, 0.0)
        sums = jnp.dot(onehot, h2_scr[...],
                       preferred_element_type=jnp.float32)
        counts = jnp.sum(onehot, axis=1, keepdims=True)
        pooled = sums / jnp.maximum(counts, 1.0)
        logits = jnp.dot(pooled, fcw_ref[...],
                         preferred_element_type=jnp.float32) + fcb_ref[...]
        mx = jnp.max(logits, axis=1, keepdims=True)
        z = logits - mx
        lse = jnp.log(jnp.sum(jnp.exp(z), axis=1, keepdims=True))
        out_ref[...] = z - lse


def kernel(x, batch_idx, attn_tensor, w_agg, b_agg,
           W1, att_src1, att_dst1, We1, att_e1, b1,
           W2, att_src2, att_dst2, We2, att_e2, b2,
           fc_w, fc_b):
    w_agg2 = jnp.reshape(w_agg.astype(jnp.float32), (1, NCH))
    b_agg2 = jnp.reshape(b_agg.astype(jnp.float32), (1, 1))
    b1_2 = jnp.reshape(b1, (1, HEADS * HID))
    b2_2 = jnp.reshape(b2, (1, HID))
    fcb2 = jnp.reshape(fc_b, (1, OUT))
    batch2 = jnp.reshape(batch_idx.astype(jnp.int32), (1, N))

    full = lambda shape: pl.BlockSpec(shape, lambda j: (0,) * len(shape))

    agg_mat, h1 = pl.pallas_call(
        _layer1_kernel,
        grid=(NJ1,),
        in_specs=[
            pl.BlockSpec((NCH, N, TJ1), lambda j: (0, 0, j)),
            full((N, F_IN)),
            full((1, NCH)),
            full((1, 1)),
            full((F_IN, HEADS * HID)),
            full((HEADS, HID)),
            full((HEADS, HID)),
            full((1, HEADS * HID)),
            full((HEADS, HID)),
            full((1, HEADS * HID)),
        ],
        out_specs=[
            pl.BlockSpec((N, TJ1), lambda j: (0, j)),
            pl.BlockSpec((TJ1, HEADS * HID), lambda j: (j, 0)),
        ],
        out_shape=[
            jax.ShapeDtypeStruct((N, N), jnp.float32),
            jax.ShapeDtypeStruct((N, HEADS * HID), jnp.float32),
        ],
        scratch_shapes=[
            pltpu.VMEM((N, HEADS * HID), jnp.float32),
            pltpu.VMEM((N, 8), jnp.float32),
            pltpu.VMEM((8, N), jnp.float32),
        ],
        compiler_params=pltpu.CompilerParams(
            dimension_semantics=("arbitrary",)),
    )(attn_tensor, x, w_agg2, b_agg2, W1, att_src1, att_dst1, We1,
      att_e1, b1_2)

    out = pl.pallas_call(
        _layer2_kernel,
        grid=(NJ2,),
        in_specs=[
            pl.BlockSpec((N, TJ2), lambda j: (0, j)),
            full((N, HEADS * HID)),
            full((HEADS * HID, HID)),
            full((1, HID)),
            full((1, HID)),
            full((1, HID)),
            full((1, HID)),
            full((1, HID)),
            full((1, N)),
            full((HID, OUT)),
            full((1, OUT)),
        ],
        out_specs=pl.BlockSpec((NG, OUT), lambda j: (0, 0)),
        out_shape=jax.ShapeDtypeStruct((NG, OUT), jnp.float32),
        scratch_shapes=[
            pltpu.VMEM((N, HID), jnp.float32),
            pltpu.VMEM((N, HID), jnp.float32),
            pltpu.VMEM((N, 8), jnp.float32),
            pltpu.VMEM((8, N), jnp.float32),
        ],
        compiler_params=pltpu.CompilerParams(
            dimension_semantics=("arbitrary",)),
    )(agg_mat, h1, W2, att_src2, att_dst2, We2, att_e2, b2_2, batch2,
      fc_w, fcb2)
    return out
